# baseline (speedup 1.0000x reference)
import jax
import jax.numpy as jnp
from jax.experimental import pallas as pl
from jax.experimental.pallas import tpu as pltpu

_CH = 512
_NBUF = 4


def _fused_router_kernel(x_hbm, w1_ref, b1_ref, w2t_ref, b2_ref, o_ref, xbufs, sems):
    n_chunks = x_hbm.shape[0] // _CH
    w2 = w2t_ref[...].T

    def start_copy(c, slot):
        pltpu.make_async_copy(
            x_hbm.at[pl.ds(c * _CH, _CH), :], xbufs.at[slot], sems.at[slot]
        ).start()

    for c in range(_NBUF):
        start_copy(c, c)

    def step(c, carry):
        slot = jax.lax.rem(c, _NBUF)
        pltpu.make_async_copy(
            x_hbm.at[pl.ds(c * _CH, _CH), :], xbufs.at[slot], sems.at[slot]
        ).wait()
        xc = xbufs[slot]
        h = jnp.dot(xc, w1_ref[...], preferred_element_type=jnp.float32)
        h = jnp.maximum(h + b1_ref[...], 0.0)
        s = jnp.dot(h, w2, preferred_element_type=jnp.float32)
        o_ref[:, pl.ds(c * _CH, _CH)] = (s + b2_ref[...]).T

        @pl.when(c + _NBUF < n_chunks)
        def _():
            start_copy(c + _NBUF, slot)

        return carry

    jax.lax.fori_loop(0, n_chunks, step, 0)


def kernel(x, W1, b1, W2, b2):
    M, K = x.shape
    H = W1.shape[1]
    G = W2.shape[1]

    out_t = pl.pallas_call(
        _fused_router_kernel,
        in_specs=[
            pl.BlockSpec(memory_space=pltpu.HBM),
            pl.BlockSpec(memory_space=pltpu.VMEM),
            pl.BlockSpec(memory_space=pltpu.VMEM),
            pl.BlockSpec(memory_space=pltpu.VMEM),
            pl.BlockSpec(memory_space=pltpu.VMEM),
        ],
        out_specs=pl.BlockSpec(memory_space=pltpu.VMEM),
        out_shape=jax.ShapeDtypeStruct((G, M), jnp.float32),
        scratch_shapes=[
            pltpu.VMEM((_NBUF, _CH, K), jnp.float32),
            pltpu.SemaphoreType.DMA((_NBUF,)),
        ],
    )(x, W1, b1.reshape(1, H), W2.T, b2.reshape(1, G))
    return out_t.T
